# Initial kernel scaffold; baseline (speedup 1.0000x reference)
import jax
import jax.numpy as jnp
from jax.experimental import pallas as pl


def kernel(x, A_w, W_query_w, keys, B_w):
    raise NotImplementedError("write your pallas kernel here")


# speedup vs baseline: 5.5227x; 5.5227x over previous
import functools

import jax
import jax.numpy as jnp
from jax.experimental import pallas as pl

NUM_EXPERTS = 64
TOP_K = 8
SCALING = 32 / 8


def _fused_body(x_ref, a_ref, wq_ref, kt_ref, b_ref, o_ref):
    xb = x_ref[...]
    z = jnp.dot(xb, a_ref[...], preferred_element_type=jnp.float32)
    q = jnp.dot(xb, wq_ref[...], preferred_element_type=jnp.float32)
    s = jnp.dot(q, kt_ref[...], preferred_element_type=jnp.float32)

    tm, ne = s.shape
    iota = jax.lax.broadcasted_iota(jnp.int32, (tm, ne), 1)
    work = s
    sel = jnp.zeros((tm, ne), jnp.bool_)
    row_max = None
    for k in range(TOP_K):
        m = jnp.max(work, axis=-1, keepdims=True)
        if k == 0:
            row_max = m
        cand = jnp.where(work == m, iota, ne)
        mi = jnp.min(cand, axis=-1, keepdims=True)
        selk = iota == mi
        sel = jnp.logical_or(sel, selk)
        work = jnp.where(selk, -jnp.inf, work)

    e = jnp.where(sel, jnp.exp(s - row_max), 0.0)
    gate = e / jnp.sum(e, axis=-1, keepdims=True)
    zg = z * gate
    o_ref[...] = jnp.dot(zg, b_ref[...], preferred_element_type=jnp.float32) * SCALING


@jax.jit
def kernel(x, A_w, W_query_w, keys, B_w):
    bsz, ssz, in_f = x.shape
    out_f = B_w.shape[1]
    t = bsz * ssz
    xf = x.reshape(t, in_f)
    keys_t = keys.T

    tm = 512
    grid = (t // tm,)
    out = pl.pallas_call(
        _fused_body,
        grid=grid,
        in_specs=[
            pl.BlockSpec((tm, in_f), lambda i: (i, 0)),
            pl.BlockSpec(A_w.shape, lambda i: (0, 0)),
            pl.BlockSpec(W_query_w.shape, lambda i: (0, 0)),
            pl.BlockSpec(keys_t.shape, lambda i: (0, 0)),
            pl.BlockSpec(B_w.shape, lambda i: (0, 0)),
        ],
        out_specs=pl.BlockSpec((tm, out_f), lambda i: (i, 0)),
        out_shape=jax.ShapeDtypeStruct((t, out_f), jnp.float32),
    )(xf, A_w, W_query_w, keys_t, B_w)
    return out.reshape(bsz, ssz, out_f)


# speedup vs baseline: 8.6687x; 1.5696x over previous
import functools

import jax
import jax.numpy as jnp
from jax.experimental import pallas as pl

NUM_EXPERTS = 64
TOP_K = 8
SCALING = 32 / 8


def _fused_body(x_ref, a_ref, wq_ref, k_ref, b_ref, o_ref):
    xb = x_ref[...]
    z = jnp.dot(xb, a_ref[...], preferred_element_type=jnp.float32)
    q = jnp.dot(xb, wq_ref[...], preferred_element_type=jnp.float32)
    st = jnp.dot(k_ref[...], q.T, preferred_element_type=jnp.float32)

    ne, tm = st.shape
    iota = jax.lax.broadcasted_iota(jnp.int32, (ne, tm), 0)
    work = st
    row_max = None
    for k in range(TOP_K):
        m = jnp.max(work, axis=0, keepdims=True)
        if k == 0:
            row_max = m
        cand = jnp.where(work == m, iota, ne)
        mi = jnp.min(cand, axis=0, keepdims=True)
        work = jnp.where(iota == mi, -jnp.inf, work)

    sel = work != st
    e = jnp.where(sel, jnp.exp(st - row_max), 0.0)
    gate_t = e / jnp.sum(e, axis=0, keepdims=True)
    zg = z * gate_t.T
    o_ref[...] = jnp.dot(zg, b_ref[...], preferred_element_type=jnp.float32) * SCALING


@jax.jit
def kernel(x, A_w, W_query_w, keys, B_w):
    bsz, ssz, in_f = x.shape
    out_f = B_w.shape[1]
    t = bsz * ssz
    xf = x.reshape(t, in_f)

    tm = 512
    grid = (t // tm,)
    out = pl.pallas_call(
        _fused_body,
        grid=grid,
        in_specs=[
            pl.BlockSpec((tm, in_f), lambda i: (i, 0)),
            pl.BlockSpec(A_w.shape, lambda i: (0, 0)),
            pl.BlockSpec(W_query_w.shape, lambda i: (0, 0)),
            pl.BlockSpec(keys.shape, lambda i: (0, 0)),
            pl.BlockSpec(B_w.shape, lambda i: (0, 0)),
        ],
        out_specs=pl.BlockSpec((tm, out_f), lambda i: (i, 0)),
        out_shape=jax.ShapeDtypeStruct((t, out_f), jnp.float32),
    )(xf, A_w, W_query_w, keys, B_w)
    return out.reshape(bsz, ssz, out_f)


# speedup vs baseline: 10.5348x; 1.2153x over previous
import functools

import jax
import jax.numpy as jnp
from jax.experimental import pallas as pl

NUM_EXPERTS = 64
TOP_K = 8
SCALING = 32 / 8


def _fused_body(x_ref, a_ref, wq_ref, k_ref, b_ref, o_ref):
    xb = x_ref[...]
    z = jnp.dot(xb, a_ref[...], preferred_element_type=jnp.float32)
    q = jnp.dot(xb, wq_ref[...], preferred_element_type=jnp.float32)
    st = jnp.dot(k_ref[...], q.T, preferred_element_type=jnp.float32)

    ne, tm = st.shape
    iota = jax.lax.broadcasted_iota(jnp.int32, (ne, tm), 0)
    work = st
    row_max = None
    for k in range(TOP_K):
        m = jnp.max(work, axis=0, keepdims=True)
        if k == 0:
            row_max = m
        cand = jnp.where(work == m, iota, ne)
        mi = jnp.min(cand, axis=0, keepdims=True)
        work = jnp.where(iota == mi, -jnp.inf, work)

    sel = work != st
    e = jnp.where(sel, jnp.exp(st - row_max), 0.0)
    gate_t = e / jnp.sum(e, axis=0, keepdims=True)
    zg = z * gate_t.T
    o_ref[...] = jnp.dot(zg, b_ref[...], preferred_element_type=jnp.float32) * SCALING


@jax.jit
def kernel(x, A_w, W_query_w, keys, B_w):
    bsz, ssz, in_f = x.shape
    out_f = B_w.shape[1]
    t = bsz * ssz
    xf = x.reshape(t, in_f)

    tm = 1024
    grid = (t // tm,)
    out = pl.pallas_call(
        _fused_body,
        grid=grid,
        in_specs=[
            pl.BlockSpec((tm, in_f), lambda i: (i, 0)),
            pl.BlockSpec(A_w.shape, lambda i: (0, 0)),
            pl.BlockSpec(W_query_w.shape, lambda i: (0, 0)),
            pl.BlockSpec(keys.shape, lambda i: (0, 0)),
            pl.BlockSpec(B_w.shape, lambda i: (0, 0)),
        ],
        out_specs=pl.BlockSpec((tm, out_f), lambda i: (i, 0)),
        out_shape=jax.ShapeDtypeStruct((t, out_f), jnp.float32),
    )(xf, A_w, W_query_w, keys, B_w)
    return out.reshape(bsz, ssz, out_f)


# speedup vs baseline: 11.2201x; 1.0651x over previous
import functools

import jax
import jax.numpy as jnp
from jax.experimental import pallas as pl

NUM_EXPERTS = 64
TOP_K = 8
SCALING = 32 / 8


def _fused_body(x_ref, a_ref, wq_ref, k_ref, b_ref, o_ref):
    xb = x_ref[...]
    z = jnp.dot(xb, a_ref[...], preferred_element_type=jnp.float32)
    q = jnp.dot(xb, wq_ref[...], preferred_element_type=jnp.float32)
    st = jnp.dot(k_ref[...], q.T, preferred_element_type=jnp.float32)

    ne, tm = st.shape
    iota = jax.lax.broadcasted_iota(jnp.int32, (ne, tm), 0)
    work = st
    row_max = None
    for k in range(TOP_K):
        m = jnp.max(work, axis=0, keepdims=True)
        if k == 0:
            row_max = m
        cand = jnp.where(work == m, iota, ne)
        mi = jnp.min(cand, axis=0, keepdims=True)
        work = jnp.where(iota == mi, -jnp.inf, work)

    sel = work != st
    e = jnp.where(sel, jnp.exp(st - row_max), 0.0)
    gate_t = e / jnp.sum(e, axis=0, keepdims=True)
    zg = z * gate_t.T
    o_ref[...] = jnp.dot(zg, b_ref[...], preferred_element_type=jnp.float32) * SCALING


@jax.jit
def kernel(x, A_w, W_query_w, keys, B_w):
    bsz, ssz, in_f = x.shape
    out_f = B_w.shape[1]
    t = bsz * ssz
    xf = x.reshape(t, in_f)

    tm = 2048
    grid = (t // tm,)
    out = pl.pallas_call(
        _fused_body,
        grid=grid,
        in_specs=[
            pl.BlockSpec((tm, in_f), lambda i: (i, 0)),
            pl.BlockSpec(A_w.shape, lambda i: (0, 0)),
            pl.BlockSpec(W_query_w.shape, lambda i: (0, 0)),
            pl.BlockSpec(keys.shape, lambda i: (0, 0)),
            pl.BlockSpec(B_w.shape, lambda i: (0, 0)),
        ],
        out_specs=pl.BlockSpec((tm, out_f), lambda i: (i, 0)),
        out_shape=jax.ShapeDtypeStruct((t, out_f), jnp.float32),
    )(xf, A_w, W_query_w, keys, B_w)
    return out.reshape(bsz, ssz, out_f)
